# baseline (speedup 1.0000x reference)
import functools

import jax
import jax.numpy as jnp
from jax import lax
from jax.experimental import pallas as pl
from jax.experimental.pallas import tpu as pltpu
from jax.experimental.pallas import tpu_sc as plsc

D = 1024
F = 4096
E = 8
K = 2
M = 128
FC = 2048
NF = F // FC
PREC = lax.Precision.DEFAULT


def _gate_body(x_ref, gw_ref, gb_ref, ti0_ref, ti1_ref, tw0_ref, tw1_ref):
    x = x_ref[...]
    logits = jnp.dot(x, gw_ref[...], preferred_element_type=jnp.float32,
                     precision=lax.Precision.DEFAULT) + gb_ref[...]
    n = x.shape[0]
    lane = lax.broadcasted_iota(jnp.int32, (n, 128), 1)
    valid = lane < E
    logits = jnp.where(valid, logits, -1e30)
    m = jnp.max(logits, axis=1, keepdims=True)
    p = jnp.where(valid, jnp.exp(logits - m), 0.0)
    probs = p / jnp.sum(p, axis=1, keepdims=True)
    v1 = jnp.max(probs, axis=1, keepdims=True)
    i1 = jnp.min(jnp.where(probs >= v1, lane, 127), axis=1, keepdims=True)
    probs2 = jnp.where(lane == i1, -1.0, probs)
    v2 = jnp.max(probs2, axis=1, keepdims=True)
    i2 = jnp.min(jnp.where(probs2 >= v2, lane, 127), axis=1, keepdims=True)
    norm = v1 + v2 + 1e-9
    ti0_ref[...] = i1
    ti1_ref[...] = i2
    tw0_ref[...] = v1 / norm
    tw1_ref[...] = v2 / norm


def _gate(flat, gate_w, gate_b):
    T = flat.shape[0]
    gw = jnp.pad(gate_w, ((0, 0), (0, 128 - E)))
    gb = jnp.pad(gate_b, (0, 128 - E)).reshape(1, 128)
    BT = 512
    grid = (T // BT,)
    out_shapes = [
        jax.ShapeDtypeStruct((T, 1), jnp.int32),
        jax.ShapeDtypeStruct((T, 1), jnp.int32),
        jax.ShapeDtypeStruct((T, 1), jnp.float32),
        jax.ShapeDtypeStruct((T, 1), jnp.float32),
    ]
    col = pl.BlockSpec((BT, 1), lambda i: (i, 0))
    return pl.pallas_call(
        _gate_body,
        grid=grid,
        in_specs=[
            pl.BlockSpec((BT, D), lambda i: (i, 0)),
            pl.BlockSpec((D, 128), lambda i: (0, 0)),
            pl.BlockSpec((1, 128), lambda i: (0, 0)),
        ],
        out_specs=[col, col, col, col],
        out_shape=out_shapes,
    )(flat, gw, gb)


def _sc_gather(table, idx):
    NR = idx.shape[0]
    NW = 32
    per_w = NR // NW
    CH = 64 if per_w % 64 == 0 else 32
    nch = per_w // CH
    mesh = plsc.VectorSubcoreMesh(core_axis_name="c", subcore_axis_name="s")

    @functools.partial(
        pl.kernel,
        mesh=mesh,
        out_type=jax.ShapeDtypeStruct((NR, D), table.dtype),
        scratch_types=[
            pltpu.VMEM((CH,), jnp.int32),
            pltpu.VMEM((CH, D), table.dtype),
            pltpu.SemaphoreType.DMA,
        ],
    )
    def k(table_hbm, idx_hbm, out_hbm, idx_v, rows_v, sem):
        wid = lax.axis_index("s") * 2 + lax.axis_index("c")

        def body(c, carry):
            base = wid * per_w + c * CH
            pltpu.sync_copy(idx_hbm.at[pl.ds(base, CH)], idx_v)
            pltpu.async_copy(table_hbm.at[idx_v], rows_v, sem).wait()
            pltpu.sync_copy(rows_v, out_hbm.at[pl.ds(base, CH)])
            return carry

        lax.fori_loop(0, nch, body, 0)

    return k(table, idx)


def _gemm_body(meta_ref, x_ref, w1_ref, b1_ref, w2_ref, b2_ref, ws_ref, y_ref):
    f = pl.program_id(0)
    b = pl.program_id(1)
    nb = pl.num_programs(1)

    @pl.when(b < meta_ref[nb])
    def _():
        xb = x_ref[...].astype(jnp.bfloat16)
        w1b = w1_ref[0].astype(jnp.bfloat16)
        h = jnp.dot(xb, w1b, preferred_element_type=jnp.float32) + b1_ref[0]
        h = 0.5 * h * (1.0 + lax.erf(h * 0.7071067811865476))
        hb = h.astype(jnp.bfloat16)
        w2b = w2_ref[0].astype(jnp.bfloat16)
        part = jnp.dot(hb, w2b, preferred_element_type=jnp.float32)

        @pl.when(f == 0)
        def _():
            y_ref[...] = (part + b2_ref[0]) * ws_ref[...]

        @pl.when(f > 0)
        def _():
            y_ref[...] = part * ws_ref[...]


def _grouped_gemm(x_slot, w1, b1, w2, b2, w_slot, meta, nb):
    NP = x_slot.shape[0]
    grid_spec = pltpu.PrefetchScalarGridSpec(
        num_scalar_prefetch=1,
        grid=(NF, nb),
        in_specs=[
            pl.BlockSpec((M, D), lambda f, b, m: (b, 0)),
            pl.BlockSpec((1, D, FC), lambda f, b, m: (m[b], 0, f)),
            pl.BlockSpec((1, 1, FC), lambda f, b, m: (m[b], 0, f)),
            pl.BlockSpec((1, FC, D), lambda f, b, m: (m[b], f, 0)),
            pl.BlockSpec((1, 1, D), lambda f, b, m: (m[b], 0, 0)),
            pl.BlockSpec((M, 1), lambda f, b, m: (b, 0)),
        ],
        out_specs=pl.BlockSpec((M, D), lambda f, b, m: (f * nb + b, 0)),
    )
    return pl.pallas_call(
        _gemm_body,
        grid_spec=grid_spec,
        out_shape=jax.ShapeDtypeStruct((NF * NP, D), jnp.float32),
    )(meta, x_slot, w1, b1.reshape(E, 1, F), w2, b2.reshape(E, 1, D), w_slot)


def _pairadd_body(a_ref, b_ref, o_ref):
    o_ref[...] = a_ref[...] + b_ref[...]


def _pairadd_body4(a_ref, b_ref, c_ref, d_ref, o_ref):
    o_ref[...] = ((a_ref[...].astype(jnp.float32) +
                   b_ref[...].astype(jnp.float32)) +
                  (c_ref[...].astype(jnp.float32) +
                   d_ref[...].astype(jnp.float32)))


def _pair_add(z, T):
    BT = 256
    nq = T // BT
    qspec = [pl.BlockSpec((BT, D), lambda i, q=q: (i + q * nq, 0))
             for q in range(4)]
    return pl.pallas_call(
        _pairadd_body4,
        grid=(nq,),
        in_specs=qspec,
        out_specs=pl.BlockSpec((BT, D), lambda i: (i, 0)),
        out_shape=jax.ShapeDtypeStruct((T, D), jnp.float32),
    )(z, z, z, z)


def kernel(x, gate_w, gate_b, w1, b1, w2, b2):
    B, S, _ = x.shape
    T = B * S
    P = T * K
    NB = P // M + E
    NP = NB * M

    flat = x.reshape(T, D)
    ti0, ti1, tw0, tw1 = _gate(flat, gate_w, gate_b)

    ke = jnp.concatenate([ti0[:, 0], ti1[:, 0]])
    wk = jnp.concatenate([tw0[:, 0], tw1[:, 0]])
    onehot = (ke[:, None] == jnp.arange(E, dtype=jnp.int32)[None, :])
    oh = onehot.astype(jnp.int32)
    counts = oh.sum(axis=0)
    nblk = (counts + M - 1) // M
    blk_off = jnp.concatenate(
        [jnp.zeros((1,), jnp.int32), jnp.cumsum(nblk)[:-1].astype(jnp.int32)])
    used_blocks = nblk.sum().astype(jnp.int32)
    padded_off = blk_off * M
    rank = (jnp.cumsum(oh, axis=0) - oh)
    myrank = jnp.sum(rank * oh, axis=1)
    pos = (padded_off[ke] + myrank).astype(jnp.int32)
    pair_payload = jnp.stack(
        [jnp.arange(P, dtype=jnp.int32) % T,
         lax.bitcast_convert_type(wk, jnp.int32)], axis=1)
    slot_init = jnp.stack(
        [jnp.arange(NP, dtype=jnp.int32) % T,
         jnp.zeros((NP,), jnp.int32)], axis=1)
    slot_payload = slot_init.at[pos].set(pair_payload)
    tok_slot = slot_payload[:, 0]
    w_slot = lax.bitcast_convert_type(
        slot_payload[:, 1], jnp.float32).reshape(NP, 1)
    barange = jnp.arange(NB, dtype=jnp.int32)
    be = jnp.clip((barange[:, None] >= blk_off[None, :]).sum(axis=1) - 1,
                  0, E - 1).astype(jnp.int32)
    meta = jnp.concatenate([be, used_blocks[None]])

    x_slot = _sc_gather(flat, tok_slot)

    y_slot = _grouped_gemm(x_slot, w1, b1, w2, b2, w_slot, meta, NB)

    pos2 = jnp.concatenate([pos, pos + NP])
    z = _sc_gather(y_slot, pos2)
    out = _pair_add(z, T)
    return out.reshape(B, S, D)


# speedup vs baseline: 1.0669x; 1.0669x over previous
import functools

import jax
import jax.numpy as jnp
from jax import lax
from jax.experimental import pallas as pl
from jax.experimental.pallas import tpu as pltpu
from jax.experimental.pallas import tpu_sc as plsc

D = 1024
F = 4096
E = 8
K = 2
M = 256
FC = 2048
NF = F // FC
PREC = lax.Precision.DEFAULT


def _gate_body(x_ref, gw_ref, gb_ref, ti0_ref, ti1_ref, tw0_ref, tw1_ref):
    x = x_ref[...]
    logits = jnp.dot(x, gw_ref[...], preferred_element_type=jnp.float32,
                     precision=lax.Precision.DEFAULT) + gb_ref[...]
    n = x.shape[0]
    lane = lax.broadcasted_iota(jnp.int32, (n, 128), 1)
    valid = lane < E
    logits = jnp.where(valid, logits, -1e30)
    m = jnp.max(logits, axis=1, keepdims=True)
    p = jnp.where(valid, jnp.exp(logits - m), 0.0)
    probs = p / jnp.sum(p, axis=1, keepdims=True)
    v1 = jnp.max(probs, axis=1, keepdims=True)
    i1 = jnp.min(jnp.where(probs >= v1, lane, 127), axis=1, keepdims=True)
    probs2 = jnp.where(lane == i1, -1.0, probs)
    v2 = jnp.max(probs2, axis=1, keepdims=True)
    i2 = jnp.min(jnp.where(probs2 >= v2, lane, 127), axis=1, keepdims=True)
    norm = v1 + v2 + 1e-9
    ti0_ref[...] = i1
    ti1_ref[...] = i2
    tw0_ref[...] = v1 / norm
    tw1_ref[...] = v2 / norm


def _gate(flat, gate_w, gate_b):
    T = flat.shape[0]
    gw = jnp.pad(gate_w, ((0, 0), (0, 128 - E)))
    gb = jnp.pad(gate_b, (0, 128 - E)).reshape(1, 128)
    BT = 512
    grid = (T // BT,)
    out_shapes = [
        jax.ShapeDtypeStruct((T, 1), jnp.int32),
        jax.ShapeDtypeStruct((T, 1), jnp.int32),
        jax.ShapeDtypeStruct((T, 1), jnp.float32),
        jax.ShapeDtypeStruct((T, 1), jnp.float32),
    ]
    col = pl.BlockSpec((BT, 1), lambda i: (i, 0))
    return pl.pallas_call(
        _gate_body,
        grid=grid,
        in_specs=[
            pl.BlockSpec((BT, D), lambda i: (i, 0)),
            pl.BlockSpec((D, 128), lambda i: (0, 0)),
            pl.BlockSpec((1, 128), lambda i: (0, 0)),
        ],
        out_specs=[col, col, col, col],
        out_shape=out_shapes,
    )(flat, gw, gb)


def _sc_gather(table, idx):
    NR = idx.shape[0]
    NW = 32
    per_w = NR // NW
    CH = 64 if per_w % 64 == 0 else 32
    nch = per_w // CH
    mesh = plsc.VectorSubcoreMesh(core_axis_name="c", subcore_axis_name="s")

    @functools.partial(
        pl.kernel,
        mesh=mesh,
        out_type=jax.ShapeDtypeStruct((NR, D), table.dtype),
        scratch_types=[
            pltpu.VMEM((CH,), jnp.int32),
            pltpu.VMEM((CH, D), table.dtype),
            pltpu.SemaphoreType.DMA,
        ],
    )
    def k(table_hbm, idx_hbm, out_hbm, idx_v, rows_v, sem):
        wid = lax.axis_index("s") * 2 + lax.axis_index("c")

        def body(c, carry):
            base = wid * per_w + c * CH
            pltpu.sync_copy(idx_hbm.at[pl.ds(base, CH)], idx_v)
            pltpu.async_copy(table_hbm.at[idx_v], rows_v, sem).wait()
            pltpu.sync_copy(rows_v, out_hbm.at[pl.ds(base, CH)])
            return carry

        lax.fori_loop(0, nch, body, 0)

    return k(table, idx)


def _gemm_body(meta_ref, x_ref, w1_ref, b1_ref, w2_ref, b2_ref, ws_ref, y_ref):
    f = pl.program_id(0)
    b = pl.program_id(1)
    nb = pl.num_programs(1)

    @pl.when(b < meta_ref[nb])
    def _():
        xb = x_ref[...].astype(jnp.bfloat16)
        w1b = w1_ref[0].astype(jnp.bfloat16)
        h = jnp.dot(xb, w1b, preferred_element_type=jnp.float32) + b1_ref[0]
        h = 0.5 * h * (1.0 + lax.erf(h * 0.7071067811865476))
        hb = h.astype(jnp.bfloat16)
        w2b = w2_ref[0].astype(jnp.bfloat16)
        part = jnp.dot(hb, w2b, preferred_element_type=jnp.float32)

        @pl.when(f == 0)
        def _():
            y_ref[...] = (part + b2_ref[0]) * ws_ref[...]

        @pl.when(f > 0)
        def _():
            y_ref[...] = part * ws_ref[...]


def _grouped_gemm(x_slot, w1, b1, w2, b2, w_slot, meta, nb):
    NP = x_slot.shape[0]
    grid_spec = pltpu.PrefetchScalarGridSpec(
        num_scalar_prefetch=1,
        grid=(NF, nb),
        in_specs=[
            pl.BlockSpec((M, D), lambda f, b, m: (b, 0)),
            pl.BlockSpec((1, D, FC), lambda f, b, m: (m[b], 0, f)),
            pl.BlockSpec((1, 1, FC), lambda f, b, m: (m[b], 0, f)),
            pl.BlockSpec((1, FC, D), lambda f, b, m: (m[b], f, 0)),
            pl.BlockSpec((1, 1, D), lambda f, b, m: (m[b], 0, 0)),
            pl.BlockSpec((M, 1), lambda f, b, m: (b, 0)),
        ],
        out_specs=pl.BlockSpec((M, D), lambda f, b, m: (f * nb + b, 0)),
    )
    return pl.pallas_call(
        _gemm_body,
        grid_spec=grid_spec,
        out_shape=jax.ShapeDtypeStruct((NF * NP, D), jnp.float32),
    )(meta, x_slot, w1, b1.reshape(E, 1, F), w2, b2.reshape(E, 1, D), w_slot)


def _pairadd_body(a_ref, b_ref, o_ref):
    o_ref[...] = a_ref[...] + b_ref[...]


def _pairadd_body4(a_ref, b_ref, c_ref, d_ref, o_ref):
    o_ref[...] = ((a_ref[...].astype(jnp.float32) +
                   b_ref[...].astype(jnp.float32)) +
                  (c_ref[...].astype(jnp.float32) +
                   d_ref[...].astype(jnp.float32)))


def _pair_add(z, T):
    BT = 256
    nq = T // BT
    qspec = [pl.BlockSpec((BT, D), lambda i, q=q: (i + q * nq, 0))
             for q in range(4)]
    return pl.pallas_call(
        _pairadd_body4,
        grid=(nq,),
        in_specs=qspec,
        out_specs=pl.BlockSpec((BT, D), lambda i: (i, 0)),
        out_shape=jax.ShapeDtypeStruct((T, D), jnp.float32),
    )(z, z, z, z)


def kernel(x, gate_w, gate_b, w1, b1, w2, b2):
    B, S, _ = x.shape
    T = B * S
    P = T * K
    NB = P // M + E
    NP = NB * M

    flat = x.reshape(T, D)
    ti0, ti1, tw0, tw1 = _gate(flat, gate_w, gate_b)

    ke = jnp.concatenate([ti0[:, 0], ti1[:, 0]])
    wk = jnp.concatenate([tw0[:, 0], tw1[:, 0]])
    onehot = (ke[:, None] == jnp.arange(E, dtype=jnp.int32)[None, :])
    oh = onehot.astype(jnp.int32)
    counts = oh.sum(axis=0)
    nblk = (counts + M - 1) // M
    blk_off = jnp.concatenate(
        [jnp.zeros((1,), jnp.int32), jnp.cumsum(nblk)[:-1].astype(jnp.int32)])
    used_blocks = nblk.sum().astype(jnp.int32)
    padded_off = blk_off * M
    rank = (jnp.cumsum(oh, axis=0) - oh)
    myrank = jnp.sum(rank * oh, axis=1)
    pos = (padded_off[ke] + myrank).astype(jnp.int32)
    pair_payload = jnp.stack(
        [jnp.arange(P, dtype=jnp.int32) % T,
         lax.bitcast_convert_type(wk, jnp.int32)], axis=1)
    slot_init = jnp.stack(
        [jnp.arange(NP, dtype=jnp.int32) % T,
         jnp.zeros((NP,), jnp.int32)], axis=1)
    slot_payload = slot_init.at[pos].set(pair_payload)
    tok_slot = slot_payload[:, 0]
    w_slot = lax.bitcast_convert_type(
        slot_payload[:, 1], jnp.float32).reshape(NP, 1)
    barange = jnp.arange(NB, dtype=jnp.int32)
    be = jnp.clip((barange[:, None] >= blk_off[None, :]).sum(axis=1) - 1,
                  0, E - 1).astype(jnp.int32)
    meta = jnp.concatenate([be, used_blocks[None]])

    x_slot = _sc_gather(flat, tok_slot)

    y_slot = _grouped_gemm(x_slot, w1, b1, w2, b2, w_slot, meta, NB)

    pos2 = jnp.concatenate([pos, pos + NP])
    z = _sc_gather(y_slot, pos2)
    out = _pair_add(z, T)
    return out.reshape(B, S, D)
